# baseline (speedup 1.0000x reference)
import functools

import jax
import jax.numpy as jnp
from jax import lax
from jax.experimental import pallas as pl
from jax.experimental.pallas import tpu as pltpu
from jax.experimental.pallas import tpu_sc as plsc

OUT = 32
GATES = ['i', 'f', 't', 'o']
CH = 128
KSUB = 4
GRP = CH * KSUB


def _sig(x):
    return jax.nn.sigmoid(x)


def _enc_body(has_c, x_ref, w0_ref, b0_ref, w1_ref, b1_ref, *outs):
    a0 = jnp.dot(x_ref[...], w0_ref[...], preferred_element_type=jnp.float32)
    a0 = a0 + b0_ref[...]
    c0 = _sig(a0[:, 0:OUT]) * jnp.tanh(a0[:, 2 * OUT:3 * OUT])
    h0 = _sig(a0[:, 3 * OUT:4 * OUT]) * jnp.tanh(c0)
    a1 = jnp.dot(h0, w1_ref[...], preferred_element_type=jnp.float32)
    a1 = a1 + b1_ref[...]
    c1 = _sig(a1[:, 0:OUT]) * jnp.tanh(a1[:, 2 * OUT:3 * OUT])
    h1 = _sig(a1[:, 3 * OUT:4 * OUT]) * jnp.tanh(c1)
    outs[0][...] = h0[:, 0:16]
    outs[1][...] = h0[:, 16:32]
    outs[2][...] = h1[:, 0:16]
    outs[3][...] = h1[:, 16:32]
    if has_c:
        outs[4][...] = c0
        outs[5][...] = c1


def _enc_call(x, w0, b0, w1, b1, has_c, blk):
    n, fin = x.shape
    grid = (n // blk,)
    const = lambda i: (0, 0)
    row = lambda i: (i, 0)
    out_shape = [jax.ShapeDtypeStruct((n, 16), jnp.float32) for _ in range(4)]
    out_specs = [pl.BlockSpec((blk, 16), row) for _ in range(4)]
    if has_c:
        out_shape += [jax.ShapeDtypeStruct((n, OUT), jnp.float32)] * 2
        out_specs += [pl.BlockSpec((blk, OUT), row)] * 2
    return pl.pallas_call(
        functools.partial(_enc_body, has_c),
        grid=grid,
        in_specs=[
            pl.BlockSpec((blk, fin), row),
            pl.BlockSpec(w0.shape, const),
            pl.BlockSpec(b0.shape, const),
            pl.BlockSpec(w1.shape, const),
            pl.BlockSpec(b1.shape, const),
        ],
        out_specs=out_specs,
        out_shape=out_shape,
    )(x, w0, b0, w1, b1)


def _dec_body(x_ref,
              s0a, s0b, s1a, s1b,
              t0a, t0b, t1a, t1b,
              cjj, cgj,
              h0a, h0b, h1a, h1b,
              c0_ref, c1_ref,
              wd0, mjj0, mgj0, hr0, bd0,
              wd1, mjj1, mgj1, hr1, bd1,
              wab, bab,
              a_out, b_out):
    rjj = 1.0 / jnp.maximum(cjj[:, 0:1], 1.0)
    rgj = 1.0 / jnp.maximum(cgj[:, 0:1], 1.0)
    m0jj = jnp.concatenate([s0a[...], s0b[...]], axis=1) * rjj
    m1jj = jnp.concatenate([s1a[...], s1b[...]], axis=1) * rjj
    m0gj = jnp.concatenate([t0a[...], t0b[...]], axis=1) * rgj
    m1gj = jnp.concatenate([t1a[...], t1b[...]], axis=1) * rgj
    h0 = jnp.concatenate([h0a[...], h0b[...]], axis=1)
    h1 = jnp.concatenate([h1a[...], h1b[...]], axis=1)

    dot = lambda a, b: jnp.dot(a, b, preferred_element_type=jnp.float32)
    acts0 = (dot(x_ref[...], wd0[...]) + dot(m0jj, mjj0[...])
             + dot(m0gj, mgj0[...]) + dot(h0, hr0[...]) + bd0[...])
    i0 = _sig(acts0[:, 0:OUT])
    f0 = _sig(acts0[:, OUT:2 * OUT])
    t0 = jnp.tanh(acts0[:, 2 * OUT:3 * OUT])
    o0 = _sig(acts0[:, 3 * OUT:4 * OUT])
    cd0 = f0 * c0_ref[...] + i0 * t0
    hd0 = o0 * jnp.tanh(cd0)

    acts1 = (dot(hd0, wd1[...]) + dot(m1jj, mjj1[...])
             + dot(m1gj, mgj1[...]) + dot(h1, hr1[...]) + bd1[...])
    i1 = _sig(acts1[:, 0:OUT])
    f1 = _sig(acts1[:, OUT:2 * OUT])
    t1 = jnp.tanh(acts1[:, 2 * OUT:3 * OUT])
    o1 = _sig(acts1[:, 3 * OUT:4 * OUT])
    cd1 = f1 * c1_ref[...] + i1 * t1
    hd1 = o1 * jnp.tanh(cd1)

    ab = dot(hd1, wab[...]) + bab[...]
    a_out[...] = ab[:, 0:16]
    b_out[...] = ab[:, 16:32]


def _dec_call(x, sums, counts, hs, c0, c1, weights, blk):
    n = x.shape[0]
    grid = (n // blk,)
    const = lambda i: (0, 0)
    row = lambda i: (i, 0)
    in_specs = [pl.BlockSpec((blk, x.shape[1]), row)]
    in_specs += [pl.BlockSpec((blk, 16), row) for _ in range(14)]
    in_specs += [pl.BlockSpec((blk, OUT), row) for _ in range(2)]
    in_specs += [pl.BlockSpec(w.shape, const) for w in weights]
    return pl.pallas_call(
        _dec_body,
        grid=grid,
        in_specs=in_specs,
        out_specs=[pl.BlockSpec((blk, 16), row)] * 2,
        out_shape=[jax.ShapeDtypeStruct((n, 16), jnp.float32)] * 2,
    )(x, *sums, *counts, *hs, c0, c1, *weights)


def _seg_kernel(nj, ng, accr, ncjj, ncgj, njj_g, ngj_g,
                sjj, djj, sgj, dgj,
                h0a, h0b, h1a, h1b, g0a, g0b, g1a, g1b,
                z16, ones16,
                o_jj0a, o_jj1a, o_gj0a, o_gj1a, c_gj,
                o_jj0b, o_jj1b, o_gj0b, o_gj1b, c_jj,
                acc, sidx, didx, rows, ones_v, sem):
    c = lax.axis_index("c")
    s = lax.axis_index("s")
    rt = accr // 16
    r0 = s * rt

    def zero_acc():
        pltpu.sync_copy(z16.at[pl.ds(r0, rt)], acc.at[pl.ds(r0, rt)])

    def seg_pass(src2, dst2, table, out, ngrp, nchunk):
        it = (ngrp + 15) // 16

        def body(i, _):
            g = i * 16 + s

            @pl.when(g < ngrp)
            def _():
                pltpu.sync_copy(src2.at[pl.ds(g * KSUB, KSUB)], sidx)
                pltpu.sync_copy(dst2.at[pl.ds(g * KSUB, KSUB)], didx)
                cps = []
                for j in range(KSUB):
                    cps.append(pltpu.async_copy(
                        table.at[sidx.at[j]], rows.at[j], sem))
                for cp in cps:
                    cp.wait()
                for j in range(KSUB):
                    pltpu.sync_copy(rows.at[j], acc.at[didx.at[j]],
                                    add=True)
            return 0

        lax.fori_loop(0, it, body, 0)
        plsc.subcore_barrier()
        pltpu.sync_copy(acc.at[pl.ds(r0, rt)], out.at[pl.ds(r0, rt)])
        zero_acc()
        plsc.subcore_barrier()

    def cnt_pass(dst2, out, ngrp):
        pltpu.sync_copy(ones16, ones_v)
        plsc.subcore_barrier()
        it = (ngrp + 15) // 16

        def body(i, _):
            g = i * 16 + s

            @pl.when(g < ngrp)
            def _():
                pltpu.sync_copy(dst2.at[pl.ds(g * KSUB, KSUB)], didx)
                for j in range(KSUB):
                    pltpu.sync_copy(ones_v, acc.at[didx.at[j]], add=True)
            return 0

        lax.fori_loop(0, it, body, 0)
        plsc.subcore_barrier()
        pltpu.sync_copy(acc.at[pl.ds(r0, rt)], out.at[pl.ds(r0, rt)])

    zero_acc()
    plsc.subcore_barrier()

    @pl.when(c == 0)
    def _():
        seg_pass(sjj, djj, h0a, o_jj0a, njj_g, ncjj)
        seg_pass(sjj, djj, h1a, o_jj1a, njj_g, ncjj)
        seg_pass(sgj, dgj, g0a, o_gj0a, ngj_g, ncgj)
        seg_pass(sgj, dgj, g1a, o_gj1a, ngj_g, ncgj)
        cnt_pass(dgj, c_gj, ngj_g)

    @pl.when(c == 1)
    def _():
        seg_pass(sjj, djj, h0b, o_jj0b, njj_g, ncjj)
        seg_pass(sjj, djj, h1b, o_jj1b, njj_g, ncjj)
        seg_pass(sgj, dgj, g0b, o_gj0b, ngj_g, ncgj)
        seg_pass(sgj, dgj, g1b, o_gj1b, ngj_g, ncgj)
        cnt_pass(djj, c_jj, njj_g)


def _seg_call(nj, ng, accr, sjj2, djj2, sgj2, dgj2, tables, gtables):
    ncjj = sjj2.shape[0]
    ncgj = sgj2.shape[0]
    njj_g = ncjj // KSUB
    ngj_g = ncgj // KSUB
    z16 = jnp.zeros((accr, 16), jnp.float32)
    ones16 = jnp.ones((CH, 16), jnp.float32)
    mesh = plsc.VectorSubcoreMesh(core_axis_name="c", subcore_axis_name="s")
    acc16 = jax.ShapeDtypeStruct((accr, 16), jnp.float32)
    fn = pl.kernel(
        functools.partial(_seg_kernel, nj, ng, accr, ncjj, ncgj,
                          njj_g, ngj_g),
        mesh=mesh,
        out_type=[acc16] * 10,
        scratch_types=[
            pltpu.VMEM_SHARED((accr, 16), jnp.float32),
            pltpu.VMEM((KSUB, CH), jnp.int32),
            pltpu.VMEM((KSUB, CH), jnp.int32),
            pltpu.VMEM((KSUB, CH, 16), jnp.float32),
            pltpu.VMEM((CH, 16), jnp.float32),
            pltpu.SemaphoreType.DMA,
        ],
        compiler_params=pltpu.CompilerParams(use_tc_tiling_on_sc=False),
    )
    return fn(sjj2, djj2, sgj2, dgj2, *tables, *gtables, z16, ones16)


def _edge_kernel(ngrp, sjj, djj, a_tab, b_tab, ev_out, rot_out,
                 sidx, didx, ga, gb, comb, sem):
    c = lax.axis_index("c")
    s = lax.axis_index("s")
    wid = s * 2 + c
    lane = lax.iota(jnp.int32, 16)
    it = (ngrp + 31) // 32

    def body(i, _):
        g = i * 32 + wid

        @pl.when(g < ngrp)
        def _():
            pltpu.sync_copy(sjj.at[pl.ds(g * KSUB, KSUB)], sidx)
            pltpu.sync_copy(djj.at[pl.ds(g * KSUB, KSUB)], didx)
            cps = []
            for j in range(KSUB):
                cps.append(pltpu.async_copy(a_tab.at[sidx.at[j]],
                                            ga.at[j], sem))
                cps.append(pltpu.async_copy(b_tab.at[didx.at[j]],
                                            gb.at[j], sem))
            for cp in cps:
                cp.wait()
            for j in range(KSUB):
                def add_row(r, _):
                    v = ga[j, r, :] + gb[j, r, :]
                    sg = 1.0 / (1.0 + jnp.exp(-v)) - 0.5
                    comb[j * CH + r, :] = jnp.where(lane == 0, v, sg)
                    return 0
                lax.fori_loop(0, CH, add_row, 0)
            pltpu.sync_copy(comb.at[:, 0:1], ev_out.at[pl.ds(g * GRP, GRP)])
            pltpu.sync_copy(comb.at[:, 1:3], rot_out.at[pl.ds(g * GRP, GRP)])
        return 0

    lax.fori_loop(0, it, body, 0)


def _edge_call(sjj2, djj2, a_tab, b_tab):
    ncjj = sjj2.shape[0]
    ngrp = ncjj // KSUB
    ep = ncjj * CH
    mesh = plsc.VectorSubcoreMesh(core_axis_name="c", subcore_axis_name="s")
    fn = pl.kernel(
        functools.partial(_edge_kernel, ngrp),
        mesh=mesh,
        out_type=[jax.ShapeDtypeStruct((ep, 1), jnp.float32),
                  jax.ShapeDtypeStruct((ep, 2), jnp.float32)],
        scratch_types=[
            pltpu.VMEM((KSUB, CH), jnp.int32),
            pltpu.VMEM((KSUB, CH), jnp.int32),
            pltpu.VMEM((KSUB, CH, 16), jnp.float32),
            pltpu.VMEM((KSUB, CH, 16), jnp.float32),
            pltpu.VMEM((GRP, 16), jnp.float32),
            pltpu.SemaphoreType.DMA,
        ],
        compiler_params=pltpu.CompilerParams(use_tc_tiling_on_sc=False),
    )
    return fn(sjj2, djj2, a_tab, b_tab)


def _gate_pack(cp, nt, extra_bias_edges):
    ws, bs = [], []
    for g in GATES:
        ws.append(cp['W_' + g][nt])
        b = cp['b_' + g][nt]
        for en in extra_bias_edges:
            b = b + cp['conv_' + g][en]['lin_l_b'][None, :]
        bs.append(b)
    return jnp.concatenate(ws, axis=1), jnp.concatenate(bs, axis=1)


def _linl_pack(cp, en):
    return jnp.concatenate(
        [cp['conv_' + g][en]['lin_l_w'] for g in GATES], axis=1)


def _linr_pack(cp, ens):
    return jnp.concatenate(
        [sum(cp['conv_' + g][en]['lin_r_w'] for en in ens) for g in GATES],
        axis=1)


def _pad_edges(e, dump_row):
    n = e.shape[1]
    npad = (-n) % GRP
    src = jnp.pad(e[0], (0, npad))
    dst = jnp.pad(e[1], (0, npad), constant_values=dump_row)
    nc = (n + npad) // CH
    return src.reshape(nc, CH), dst.reshape(nc, CH)


def kernel(x_joint, x_grain, params, edge_index_jj, edge_index_jg,
           edge_index_gj):
    p = params
    nj = x_joint.shape[0]
    ng = x_grain.shape[0]
    ejj = edge_index_jj.shape[1]
    accr = ((nj + 16 * CH) // (16 * CH)) * 16 * CH
    xj = jnp.pad(x_joint, ((0, accr - nj), (0, 0)))

    we0j, be0j = _gate_pack(p['enc0'], 'joint', ['jj', 'gj'])
    we1j, be1j = _gate_pack(p['enc1'], 'joint', ['jj', 'gj'])
    we0g, be0g = _gate_pack(p['enc0'], 'grain', ['jg'])
    we1g, be1g = _gate_pack(p['enc1'], 'grain', ['jg'])
    h0a, h0b, h1a, h1b, c0, c1 = _enc_call(
        xj, we0j, be0j, we1j, be1j, True, 2048)
    g0a, g0b, g1a, g1b = _enc_call(
        x_grain, we0g, be0g, we1g, be1g, False, 2000)

    sjj2, djj2 = _pad_edges(edge_index_jj, nj)
    sgj2, dgj2 = _pad_edges(edge_index_gj, nj)
    (s_jj0a, s_jj1a, s_gj0a, s_gj1a, c_gj,
     s_jj0b, s_jj1b, s_gj0b, s_gj1b, c_jj) = _seg_call(
        nj, ng, accr, sjj2, djj2, sgj2, dgj2,
        (h0a, h0b, h1a, h1b), (g0a, g0b, g1a, g1b))

    wd0, bd0 = _gate_pack(p['dec0'], 'joint', ['jj', 'gj'])
    wd1, bd1 = _gate_pack(p['dec1'], 'joint', ['jj', 'gj'])
    mjj0 = _linl_pack(p['dec0'], 'jj')
    mgj0 = _linl_pack(p['dec0'], 'gj')
    hr0 = _linr_pack(p['dec0'], ['jj', 'gj'])
    mjj1 = _linl_pack(p['dec1'], 'jj')
    mgj1 = _linl_pack(p['dec1'], 'gj')
    hr1 = _linr_pack(p['dec1'], ['jj', 'gj'])
    za = jnp.zeros((OUT, 13), jnp.float32)
    wab = jnp.concatenate([p['lin2_w'][:OUT], p['lin1_w'][:OUT], za,
                           p['lin2_w'][OUT:], p['lin1_w'][OUT:], za], axis=1)
    bab = jnp.concatenate([p['lin2_b'], p['lin1_b'],
                           jnp.zeros((29,), jnp.float32)])[None, :]
    a_tab, b_tab = _dec_call(
        xj,
        [s_jj0a, s_jj0b, s_jj1a, s_jj1b,
         s_gj0a, s_gj0b, s_gj1a, s_gj1b],
        [c_jj, c_gj],
        [h0a, h0b, h1a, h1b], c0, c1,
        [wd0, mjj0, mgj0, hr0, bd0, wd1, mjj1, mgj1, hr1, bd1, wab, bab],
        2048)

    ev, rot = _edge_call(sjj2, djj2, a_tab, b_tab)
    edge_event = ev.reshape(-1)[:ejj]
    edge_rotation = rot[:ejj]
    return edge_event, edge_rotation


# speedup vs baseline: 2.1387x; 2.1387x over previous
import functools

import jax
import jax.numpy as jnp
from jax import lax
from jax.experimental import pallas as pl
from jax.experimental.pallas import tpu as pltpu
from jax.experimental.pallas import tpu_sc as plsc

OUT = 32
GATES = ['i', 'f', 't', 'o']
CH = 128
KSUB = 4
GRP = CH * KSUB


def _sig(x):
    return jax.nn.sigmoid(x)


def _enc_body(has_c, x_ref, w0_ref, b0_ref, w1_ref, b1_ref, *outs):
    a0 = jnp.dot(x_ref[...], w0_ref[...], preferred_element_type=jnp.float32)
    a0 = a0 + b0_ref[...]
    c0 = _sig(a0[:, 0:OUT]) * jnp.tanh(a0[:, 2 * OUT:3 * OUT])
    h0 = _sig(a0[:, 3 * OUT:4 * OUT]) * jnp.tanh(c0)
    a1 = jnp.dot(h0, w1_ref[...], preferred_element_type=jnp.float32)
    a1 = a1 + b1_ref[...]
    c1 = _sig(a1[:, 0:OUT]) * jnp.tanh(a1[:, 2 * OUT:3 * OUT])
    h1 = _sig(a1[:, 3 * OUT:4 * OUT]) * jnp.tanh(c1)
    outs[0][...] = h0[:, 0:16]
    outs[1][...] = h0[:, 16:32]
    outs[2][...] = h1[:, 0:16]
    outs[3][...] = h1[:, 16:32]
    if has_c:
        outs[4][...] = c0
        outs[5][...] = c1


def _enc_call(x, w0, b0, w1, b1, has_c, blk):
    n, fin = x.shape
    grid = (n // blk,)
    const = lambda i: (0, 0)
    row = lambda i: (i, 0)
    out_shape = [jax.ShapeDtypeStruct((n, 16), jnp.float32) for _ in range(4)]
    out_specs = [pl.BlockSpec((blk, 16), row) for _ in range(4)]
    if has_c:
        out_shape += [jax.ShapeDtypeStruct((n, OUT), jnp.float32)] * 2
        out_specs += [pl.BlockSpec((blk, OUT), row)] * 2
    return pl.pallas_call(
        functools.partial(_enc_body, has_c),
        grid=grid,
        in_specs=[
            pl.BlockSpec((blk, fin), row),
            pl.BlockSpec(w0.shape, const),
            pl.BlockSpec(b0.shape, const),
            pl.BlockSpec(w1.shape, const),
            pl.BlockSpec(b1.shape, const),
        ],
        out_specs=out_specs,
        out_shape=out_shape,
    )(x, w0, b0, w1, b1)


def _dec_body(x_ref,
              s0a, s0b, s1a, s1b,
              t0a, t0b, t1a, t1b,
              cjj, cgj,
              h0a, h0b, h1a, h1b,
              c0_ref, c1_ref,
              wd0, mjj0, mgj0, hr0, bd0,
              wd1, mjj1, mgj1, hr1, bd1,
              wab, bab,
              a_out, b_out):
    rjj = 1.0 / jnp.maximum(cjj[:, 0:1], 1.0)
    rgj = 1.0 / jnp.maximum(cgj[:, 0:1], 1.0)
    m0jj = jnp.concatenate([s0a[...], s0b[...]], axis=1) * rjj
    m1jj = jnp.concatenate([s1a[...], s1b[...]], axis=1) * rjj
    m0gj = jnp.concatenate([t0a[...], t0b[...]], axis=1) * rgj
    m1gj = jnp.concatenate([t1a[...], t1b[...]], axis=1) * rgj
    h0 = jnp.concatenate([h0a[...], h0b[...]], axis=1)
    h1 = jnp.concatenate([h1a[...], h1b[...]], axis=1)

    dot = lambda a, b: jnp.dot(a, b, preferred_element_type=jnp.float32)
    acts0 = (dot(x_ref[...], wd0[...]) + dot(m0jj, mjj0[...])
             + dot(m0gj, mgj0[...]) + dot(h0, hr0[...]) + bd0[...])
    i0 = _sig(acts0[:, 0:OUT])
    f0 = _sig(acts0[:, OUT:2 * OUT])
    t0 = jnp.tanh(acts0[:, 2 * OUT:3 * OUT])
    o0 = _sig(acts0[:, 3 * OUT:4 * OUT])
    cd0 = f0 * c0_ref[...] + i0 * t0
    hd0 = o0 * jnp.tanh(cd0)

    acts1 = (dot(hd0, wd1[...]) + dot(m1jj, mjj1[...])
             + dot(m1gj, mgj1[...]) + dot(h1, hr1[...]) + bd1[...])
    i1 = _sig(acts1[:, 0:OUT])
    f1 = _sig(acts1[:, OUT:2 * OUT])
    t1 = jnp.tanh(acts1[:, 2 * OUT:3 * OUT])
    o1 = _sig(acts1[:, 3 * OUT:4 * OUT])
    cd1 = f1 * c1_ref[...] + i1 * t1
    hd1 = o1 * jnp.tanh(cd1)

    ab = dot(hd1, wab[...]) + bab[...]
    a_out[...] = ab[:, 0:16]
    b_out[...] = ab[:, 16:32]


def _dec_call(x, sums, counts, hs, c0, c1, weights, blk):
    n = x.shape[0]
    grid = (n // blk,)
    const = lambda i: (0, 0)
    row = lambda i: (i, 0)
    in_specs = [pl.BlockSpec((blk, x.shape[1]), row)]
    in_specs += [pl.BlockSpec((blk, 16), row) for _ in range(14)]
    in_specs += [pl.BlockSpec((blk, OUT), row) for _ in range(2)]
    in_specs += [pl.BlockSpec(w.shape, const) for w in weights]
    return pl.pallas_call(
        _dec_body,
        grid=grid,
        in_specs=in_specs,
        out_specs=[pl.BlockSpec((blk, 16), row)] * 2,
        out_shape=[jax.ShapeDtypeStruct((n, 16), jnp.float32)] * 2,
    )(x, *sums, *counts, *hs, c0, c1, *weights)


def _seg_kernel(nj, ng, accr, ncjj, ncgj, njj_g, ngj_g,
                sjj, djj, sgj, dgj,
                h0a, h0b, h1a, h1b, g0a, g0b, g1a, g1b,
                z16, ones16,
                o_jj0a, o_jj1a, o_gj0a, o_gj1a, c_gj,
                o_jj0b, o_jj1b, o_gj0b, o_gj1b, c_jj,
                acc, sidx, didx, rows, ones_v, sem):
    c = lax.axis_index("c")
    s = lax.axis_index("s")
    rt = accr // 16
    r0 = s * rt

    def zero_acc():
        pltpu.sync_copy(z16.at[pl.ds(r0, rt)], acc.at[pl.ds(r0, rt)])

    def seg_pass(src2, dst2, table, out, ngrp, nchunk):
        it = (ngrp + 15) // 16

        def body(i, _):
            g = i * 16 + s

            @pl.when(g < ngrp)
            def _():
                pltpu.sync_copy(src2.at[pl.ds(g * KSUB, KSUB)], sidx)
                pltpu.sync_copy(dst2.at[pl.ds(g * KSUB, KSUB)], didx)
                cps = []
                for j in range(KSUB):
                    cps.append(pltpu.async_copy(
                        table.at[sidx.at[j]], rows.at[j], sem))
                for cp in cps:
                    cp.wait()
                for j in range(KSUB):
                    pltpu.sync_copy(rows.at[j], acc.at[didx.at[j]],
                                    add=True)
            return 0

        lax.fori_loop(0, it, body, 0)
        plsc.subcore_barrier()
        pltpu.sync_copy(acc.at[pl.ds(r0, rt)], out.at[pl.ds(r0, rt)])
        zero_acc()
        plsc.subcore_barrier()

    def cnt_pass(dst2, out, ngrp):
        pltpu.sync_copy(ones16, ones_v)
        plsc.subcore_barrier()
        it = (ngrp + 15) // 16

        def body(i, _):
            g = i * 16 + s

            @pl.when(g < ngrp)
            def _():
                pltpu.sync_copy(dst2.at[pl.ds(g * KSUB, KSUB)], didx)
                for j in range(KSUB):
                    pltpu.sync_copy(ones_v, acc.at[didx.at[j]], add=True)
            return 0

        lax.fori_loop(0, it, body, 0)
        plsc.subcore_barrier()
        pltpu.sync_copy(acc.at[pl.ds(r0, rt)], out.at[pl.ds(r0, rt)])

    zero_acc()
    plsc.subcore_barrier()

    @pl.when(c == 0)
    def _():
        seg_pass(sjj, djj, h0a, o_jj0a, njj_g, ncjj)
        seg_pass(sjj, djj, h1a, o_jj1a, njj_g, ncjj)
        seg_pass(sgj, dgj, g0a, o_gj0a, ngj_g, ncgj)
        seg_pass(sgj, dgj, g1a, o_gj1a, ngj_g, ncgj)
        cnt_pass(dgj, c_gj, ngj_g)

    @pl.when(c == 1)
    def _():
        seg_pass(sjj, djj, h0b, o_jj0b, njj_g, ncjj)
        seg_pass(sjj, djj, h1b, o_jj1b, njj_g, ncjj)
        seg_pass(sgj, dgj, g0b, o_gj0b, ngj_g, ncgj)
        seg_pass(sgj, dgj, g1b, o_gj1b, ngj_g, ncgj)
        cnt_pass(djj, c_jj, njj_g)


def _seg_call(nj, ng, accr, sjj2, djj2, sgj2, dgj2, tables, gtables):
    ncjj = sjj2.shape[0]
    ncgj = sgj2.shape[0]
    njj_g = ncjj // KSUB
    ngj_g = ncgj // KSUB
    z16 = jnp.zeros((accr, 16), jnp.float32)
    ones16 = jnp.ones((CH, 16), jnp.float32)
    mesh = plsc.VectorSubcoreMesh(core_axis_name="c", subcore_axis_name="s")
    acc16 = jax.ShapeDtypeStruct((accr, 16), jnp.float32)
    fn = pl.kernel(
        functools.partial(_seg_kernel, nj, ng, accr, ncjj, ncgj,
                          njj_g, ngj_g),
        mesh=mesh,
        out_type=[acc16] * 10,
        scratch_types=[
            pltpu.VMEM_SHARED((accr, 16), jnp.float32),
            pltpu.VMEM((KSUB, CH), jnp.int32),
            pltpu.VMEM((KSUB, CH), jnp.int32),
            pltpu.VMEM((KSUB, CH, 16), jnp.float32),
            pltpu.VMEM((CH, 16), jnp.float32),
            pltpu.SemaphoreType.DMA,
        ],
        compiler_params=pltpu.CompilerParams(use_tc_tiling_on_sc=False),
    )
    return fn(sjj2, djj2, sgj2, dgj2, *tables, *gtables, z16, ones16)


def _edge_kernel(ngrp, sjj, djj, a_tab, b_tab, comb_out,
                 sidx, didx, ga, gb, comb, sem):
    c = lax.axis_index("c")
    s = lax.axis_index("s")
    wid = s * 2 + c
    lane = lax.iota(jnp.int32, 16)
    it = (ngrp + 31) // 32

    def body(i, _):
        g = i * 32 + wid

        @pl.when(g < ngrp)
        def _():
            pltpu.sync_copy(sjj.at[pl.ds(g * KSUB, KSUB)], sidx)
            pltpu.sync_copy(djj.at[pl.ds(g * KSUB, KSUB)], didx)
            cps = []
            for j in range(KSUB):
                cps.append(pltpu.async_copy(a_tab.at[sidx.at[j]],
                                            ga.at[j], sem))
                cps.append(pltpu.async_copy(b_tab.at[didx.at[j]],
                                            gb.at[j], sem))
            for cp in cps:
                cp.wait()
            for j in range(KSUB):
                def add_row(r, _):
                    v = ga[j, r, :] + gb[j, r, :]
                    sg = 1.0 / (1.0 + jnp.exp(-v)) - 0.5
                    comb[j * CH + r, :] = jnp.where(lane == 0, v, sg)
                    return 0
                lax.fori_loop(0, CH, add_row, 0)
            pltpu.sync_copy(comb, comb_out.at[pl.ds(g * GRP, GRP)])
        return 0

    lax.fori_loop(0, it, body, 0)


def _edge_call(sjj2, djj2, a_tab, b_tab):
    ncjj = sjj2.shape[0]
    ngrp = ncjj // KSUB
    ep = ncjj * CH
    mesh = plsc.VectorSubcoreMesh(core_axis_name="c", subcore_axis_name="s")
    fn = pl.kernel(
        functools.partial(_edge_kernel, ngrp),
        mesh=mesh,
        out_type=jax.ShapeDtypeStruct((ep, 16), jnp.float32),
        scratch_types=[
            pltpu.VMEM((KSUB, CH), jnp.int32),
            pltpu.VMEM((KSUB, CH), jnp.int32),
            pltpu.VMEM((KSUB, CH, 16), jnp.float32),
            pltpu.VMEM((KSUB, CH, 16), jnp.float32),
            pltpu.VMEM((GRP, 16), jnp.float32),
            pltpu.SemaphoreType.DMA,
        ],
        compiler_params=pltpu.CompilerParams(use_tc_tiling_on_sc=False),
    )
    return fn(sjj2, djj2, a_tab, b_tab)


def _gate_pack(cp, nt, extra_bias_edges):
    ws, bs = [], []
    for g in GATES:
        ws.append(cp['W_' + g][nt])
        b = cp['b_' + g][nt]
        for en in extra_bias_edges:
            b = b + cp['conv_' + g][en]['lin_l_b'][None, :]
        bs.append(b)
    return jnp.concatenate(ws, axis=1), jnp.concatenate(bs, axis=1)


def _linl_pack(cp, en):
    return jnp.concatenate(
        [cp['conv_' + g][en]['lin_l_w'] for g in GATES], axis=1)


def _linr_pack(cp, ens):
    return jnp.concatenate(
        [sum(cp['conv_' + g][en]['lin_r_w'] for en in ens) for g in GATES],
        axis=1)


def _pad_edges(e, dump_row, grp=GRP):
    n = e.shape[1]
    npad = (-n) % grp
    src = jnp.pad(e[0], (0, npad))
    dst = jnp.pad(e[1], (0, npad), constant_values=dump_row)
    nc = (n + npad) // CH
    return src.reshape(nc, CH), dst.reshape(nc, CH)


def kernel(x_joint, x_grain, params, edge_index_jj, edge_index_jg,
           edge_index_gj):
    p = params
    nj = x_joint.shape[0]
    ng = x_grain.shape[0]
    ejj = edge_index_jj.shape[1]
    accr = ((nj + 16 * CH) // (16 * CH)) * 16 * CH
    xj = jnp.pad(x_joint, ((0, accr - nj), (0, 0)))

    we0j, be0j = _gate_pack(p['enc0'], 'joint', ['jj', 'gj'])
    we1j, be1j = _gate_pack(p['enc1'], 'joint', ['jj', 'gj'])
    we0g, be0g = _gate_pack(p['enc0'], 'grain', ['jg'])
    we1g, be1g = _gate_pack(p['enc1'], 'grain', ['jg'])
    h0a, h0b, h1a, h1b, c0, c1 = _enc_call(
        xj, we0j, be0j, we1j, be1j, True, 2048)
    g0a, g0b, g1a, g1b = _enc_call(
        x_grain, we0g, be0g, we1g, be1g, False, 2000)

    sjj2, djj2 = _pad_edges(edge_index_jj, nj)
    sgj2, dgj2 = _pad_edges(edge_index_gj, nj)
    (s_jj0a, s_jj1a, s_gj0a, s_gj1a, c_gj,
     s_jj0b, s_jj1b, s_gj0b, s_gj1b, c_jj) = _seg_call(
        nj, ng, accr, sjj2, djj2, sgj2, dgj2,
        (h0a, h0b, h1a, h1b), (g0a, g0b, g1a, g1b))

    wd0, bd0 = _gate_pack(p['dec0'], 'joint', ['jj', 'gj'])
    wd1, bd1 = _gate_pack(p['dec1'], 'joint', ['jj', 'gj'])
    mjj0 = _linl_pack(p['dec0'], 'jj')
    mgj0 = _linl_pack(p['dec0'], 'gj')
    hr0 = _linr_pack(p['dec0'], ['jj', 'gj'])
    mjj1 = _linl_pack(p['dec1'], 'jj')
    mgj1 = _linl_pack(p['dec1'], 'gj')
    hr1 = _linr_pack(p['dec1'], ['jj', 'gj'])
    za = jnp.zeros((OUT, 13), jnp.float32)
    wab = jnp.concatenate([p['lin2_w'][:OUT], p['lin1_w'][:OUT], za,
                           p['lin2_w'][OUT:], p['lin1_w'][OUT:], za], axis=1)
    bab = jnp.concatenate([p['lin2_b'], p['lin1_b'],
                           jnp.zeros((29,), jnp.float32)])[None, :]
    a_tab, b_tab = _dec_call(
        xj,
        [s_jj0a, s_jj0b, s_jj1a, s_jj1b,
         s_gj0a, s_gj0b, s_gj1a, s_gj1b],
        [c_jj, c_gj],
        [h0a, h0b, h1a, h1b], c0, c1,
        [wd0, mjj0, mgj0, hr0, bd0, wd1, mjj1, mgj1, hr1, bd1, wab, bab],
        2048)

    comb = _edge_call(sjj2, djj2, a_tab, b_tab)
    edge_event = comb[:ejj, 0]
    edge_rotation = comb[:ejj, 1:3]
    return edge_event, edge_rotation


# speedup vs baseline: 2.1993x; 1.0283x over previous
import functools

import jax
import jax.numpy as jnp
from jax import lax
from jax.experimental import pallas as pl
from jax.experimental.pallas import tpu as pltpu
from jax.experimental.pallas import tpu_sc as plsc

OUT = 32
GATES = ['i', 'f', 't', 'o']
CH = 128
KSUB = 4
GRP = CH * KSUB


def _sig(x):
    return jax.nn.sigmoid(x)


def _enc_body(has_c, x_ref, w0_ref, b0_ref, w1_ref, b1_ref, *outs):
    a0 = jnp.dot(x_ref[...], w0_ref[...], preferred_element_type=jnp.float32)
    a0 = a0 + b0_ref[...]
    c0 = _sig(a0[:, 0:OUT]) * jnp.tanh(a0[:, 2 * OUT:3 * OUT])
    h0 = _sig(a0[:, 3 * OUT:4 * OUT]) * jnp.tanh(c0)
    a1 = jnp.dot(h0, w1_ref[...], preferred_element_type=jnp.float32)
    a1 = a1 + b1_ref[...]
    c1 = _sig(a1[:, 0:OUT]) * jnp.tanh(a1[:, 2 * OUT:3 * OUT])
    h1 = _sig(a1[:, 3 * OUT:4 * OUT]) * jnp.tanh(c1)
    outs[0][...] = h0[:, 0:16]
    outs[1][...] = h0[:, 16:32]
    outs[2][...] = h1[:, 0:16]
    outs[3][...] = h1[:, 16:32]
    if has_c:
        outs[4][...] = c0
        outs[5][...] = c1


def _enc_call(x, w0, b0, w1, b1, has_c, blk):
    n, fin = x.shape
    grid = (n // blk,)
    const = lambda i: (0, 0)
    row = lambda i: (i, 0)
    out_shape = [jax.ShapeDtypeStruct((n, 16), jnp.float32) for _ in range(4)]
    out_specs = [pl.BlockSpec((blk, 16), row) for _ in range(4)]
    if has_c:
        out_shape += [jax.ShapeDtypeStruct((n, OUT), jnp.float32)] * 2
        out_specs += [pl.BlockSpec((blk, OUT), row)] * 2
    return pl.pallas_call(
        functools.partial(_enc_body, has_c),
        grid=grid,
        in_specs=[
            pl.BlockSpec((blk, fin), row),
            pl.BlockSpec(w0.shape, const),
            pl.BlockSpec(b0.shape, const),
            pl.BlockSpec(w1.shape, const),
            pl.BlockSpec(b1.shape, const),
        ],
        out_specs=out_specs,
        out_shape=out_shape,
    )(x, w0, b0, w1, b1)


def _dec_body(x_ref,
              s0a, s0b, s1a, s1b,
              t0a, t0b, t1a, t1b,
              cjj, cgj,
              h0a, h0b, h1a, h1b,
              c0_ref, c1_ref,
              wd0, mjj0, mgj0, hr0, bd0,
              wd1, mjj1, mgj1, hr1, bd1,
              wab, bab,
              a_out, b_out):
    rjj = 1.0 / jnp.maximum(cjj[:, 0:1], 1.0)
    rgj = 1.0 / jnp.maximum(cgj[:, 0:1], 1.0)
    m0jj = jnp.concatenate([s0a[...], s0b[...]], axis=1) * rjj
    m1jj = jnp.concatenate([s1a[...], s1b[...]], axis=1) * rjj
    m0gj = jnp.concatenate([t0a[...], t0b[...]], axis=1) * rgj
    m1gj = jnp.concatenate([t1a[...], t1b[...]], axis=1) * rgj
    h0 = jnp.concatenate([h0a[...], h0b[...]], axis=1)
    h1 = jnp.concatenate([h1a[...], h1b[...]], axis=1)

    dot = lambda a, b: jnp.dot(a, b, preferred_element_type=jnp.float32)
    acts0 = (dot(x_ref[...], wd0[...]) + dot(m0jj, mjj0[...])
             + dot(m0gj, mgj0[...]) + dot(h0, hr0[...]) + bd0[...])
    i0 = _sig(acts0[:, 0:OUT])
    f0 = _sig(acts0[:, OUT:2 * OUT])
    t0 = jnp.tanh(acts0[:, 2 * OUT:3 * OUT])
    o0 = _sig(acts0[:, 3 * OUT:4 * OUT])
    cd0 = f0 * c0_ref[...] + i0 * t0
    hd0 = o0 * jnp.tanh(cd0)

    acts1 = (dot(hd0, wd1[...]) + dot(m1jj, mjj1[...])
             + dot(m1gj, mgj1[...]) + dot(h1, hr1[...]) + bd1[...])
    i1 = _sig(acts1[:, 0:OUT])
    f1 = _sig(acts1[:, OUT:2 * OUT])
    t1 = jnp.tanh(acts1[:, 2 * OUT:3 * OUT])
    o1 = _sig(acts1[:, 3 * OUT:4 * OUT])
    cd1 = f1 * c1_ref[...] + i1 * t1
    hd1 = o1 * jnp.tanh(cd1)

    ab = dot(hd1, wab[...]) + bab[...]
    a_out[...] = ab[:, 0:16]
    b_out[...] = ab[:, 16:32]


def _dec_call(x, sums, counts, hs, c0, c1, weights, blk):
    n = x.shape[0]
    grid = (n // blk,)
    const = lambda i: (0, 0)
    row = lambda i: (i, 0)
    in_specs = [pl.BlockSpec((blk, x.shape[1]), row)]
    in_specs += [pl.BlockSpec((blk, 16), row) for _ in range(14)]
    in_specs += [pl.BlockSpec((blk, OUT), row) for _ in range(2)]
    in_specs += [pl.BlockSpec(w.shape, const) for w in weights]
    return pl.pallas_call(
        _dec_body,
        grid=grid,
        in_specs=in_specs,
        out_specs=[pl.BlockSpec((blk, 16), row)] * 2,
        out_shape=[jax.ShapeDtypeStruct((n, 16), jnp.float32)] * 2,
    )(x, *sums, *counts, *hs, c0, c1, *weights)


def _seg_kernel(nj, ng, accr, ncjj, ncgj, njj_g, ngj_g,
                sjj, djj, sgj, dgj,
                h0a, h0b, h1a, h1b, g0a, g0b, g1a, g1b,
                z16, ones16,
                o_jj0a, o_jj1a, o_gj0a, o_gj1a, c_gj,
                o_jj0b, o_jj1b, o_gj0b, o_gj1b, c_jj,
                acc, sidx, didx, rows, ones_v, sem):
    c = lax.axis_index("c")
    s = lax.axis_index("s")
    rt = accr // 16
    r0 = s * rt

    def zero_acc():
        pltpu.sync_copy(z16.at[pl.ds(r0, rt)], acc.at[pl.ds(r0, rt)])

    def seg_pass(src2, dst2, table, out, ngrp, nchunk):
        it = (ngrp + 15) // 16

        def body(i, _):
            g = i * 16 + s

            @pl.when(g < ngrp)
            def _():
                pltpu.sync_copy(src2.at[pl.ds(g * KSUB, KSUB)], sidx)
                pltpu.sync_copy(dst2.at[pl.ds(g * KSUB, KSUB)], didx)
                cps = []
                for j in range(KSUB):
                    cps.append(pltpu.async_copy(
                        table.at[sidx.at[j]], rows.at[j], sem))
                for cp in cps:
                    cp.wait()
                scs = []
                for j in range(KSUB):
                    scs.append(pltpu.async_copy(
                        rows.at[j], acc.at[didx.at[j]], sem, add=True))
                for cp in scs:
                    cp.wait()
            return 0

        lax.fori_loop(0, it, body, 0)
        plsc.subcore_barrier()
        pltpu.sync_copy(acc.at[pl.ds(r0, rt)], out.at[pl.ds(r0, rt)])
        zero_acc()
        plsc.subcore_barrier()

    def cnt_pass(dst2, out, ngrp):
        pltpu.sync_copy(ones16, ones_v)
        plsc.subcore_barrier()
        it = (ngrp + 15) // 16

        def body(i, _):
            g = i * 16 + s

            @pl.when(g < ngrp)
            def _():
                pltpu.sync_copy(dst2.at[pl.ds(g * KSUB, KSUB)], didx)
                scs = []
                for j in range(KSUB):
                    scs.append(pltpu.async_copy(
                        ones_v, acc.at[didx.at[j]], sem, add=True))
                for cp in scs:
                    cp.wait()
            return 0

        lax.fori_loop(0, it, body, 0)
        plsc.subcore_barrier()
        pltpu.sync_copy(acc.at[pl.ds(r0, rt)], out.at[pl.ds(r0, rt)])

    zero_acc()
    plsc.subcore_barrier()

    @pl.when(c == 0)
    def _():
        seg_pass(sjj, djj, h0a, o_jj0a, njj_g, ncjj)
        seg_pass(sjj, djj, h1a, o_jj1a, njj_g, ncjj)
        seg_pass(sgj, dgj, g0a, o_gj0a, ngj_g, ncgj)
        seg_pass(sgj, dgj, g1a, o_gj1a, ngj_g, ncgj)
        cnt_pass(dgj, c_gj, ngj_g)

    @pl.when(c == 1)
    def _():
        seg_pass(sjj, djj, h0b, o_jj0b, njj_g, ncjj)
        seg_pass(sjj, djj, h1b, o_jj1b, njj_g, ncjj)
        seg_pass(sgj, dgj, g0b, o_gj0b, ngj_g, ncgj)
        seg_pass(sgj, dgj, g1b, o_gj1b, ngj_g, ncgj)
        cnt_pass(djj, c_jj, njj_g)


def _seg_call(nj, ng, accr, sjj2, djj2, sgj2, dgj2, tables, gtables):
    ncjj = sjj2.shape[0]
    ncgj = sgj2.shape[0]
    njj_g = ncjj // KSUB
    ngj_g = ncgj // KSUB
    z16 = jnp.zeros((accr, 16), jnp.float32)
    ones16 = jnp.ones((CH, 16), jnp.float32)
    mesh = plsc.VectorSubcoreMesh(core_axis_name="c", subcore_axis_name="s")
    acc16 = jax.ShapeDtypeStruct((accr, 16), jnp.float32)
    fn = pl.kernel(
        functools.partial(_seg_kernel, nj, ng, accr, ncjj, ncgj,
                          njj_g, ngj_g),
        mesh=mesh,
        out_type=[acc16] * 10,
        scratch_types=[
            pltpu.VMEM_SHARED((accr, 16), jnp.float32),
            pltpu.VMEM((KSUB, CH), jnp.int32),
            pltpu.VMEM((KSUB, CH), jnp.int32),
            pltpu.VMEM((KSUB, CH, 16), jnp.float32),
            pltpu.VMEM((CH, 16), jnp.float32),
            pltpu.SemaphoreType.DMA,
        ],
        compiler_params=pltpu.CompilerParams(use_tc_tiling_on_sc=False),
    )
    return fn(sjj2, djj2, sgj2, dgj2, *tables, *gtables, z16, ones16)


def _edge_kernel(ngrp, sjj, djj, a_tab, b_tab, comb_out,
                 sidx, didx, ga, gb, comb, sem):
    c = lax.axis_index("c")
    s = lax.axis_index("s")
    wid = s * 2 + c
    lane = lax.iota(jnp.int32, 16)
    it = (ngrp + 31) // 32

    def body(i, _):
        g = i * 32 + wid

        @pl.when(g < ngrp)
        def _():
            pltpu.sync_copy(sjj.at[pl.ds(g * KSUB, KSUB)], sidx)
            pltpu.sync_copy(djj.at[pl.ds(g * KSUB, KSUB)], didx)
            cps = []
            for j in range(KSUB):
                cps.append(pltpu.async_copy(a_tab.at[sidx.at[j]],
                                            ga.at[j], sem))
                cps.append(pltpu.async_copy(b_tab.at[didx.at[j]],
                                            gb.at[j], sem))
            for cp in cps:
                cp.wait()
            for j in range(KSUB):
                def add_row(r, _):
                    v = ga[j, r, :] + gb[j, r, :]
                    sg = 1.0 / (1.0 + jnp.exp(-v)) - 0.5
                    comb[j * CH + r, :] = jnp.where(lane == 0, v, sg)
                    return 0
                lax.fori_loop(0, CH, add_row, 0)
            pltpu.sync_copy(comb, comb_out.at[pl.ds(g * GRP, GRP)])
        return 0

    lax.fori_loop(0, it, body, 0)


def _edge_call(sjj2, djj2, a_tab, b_tab):
    ncjj = sjj2.shape[0]
    ngrp = ncjj // KSUB
    ep = ncjj * CH
    mesh = plsc.VectorSubcoreMesh(core_axis_name="c", subcore_axis_name="s")
    fn = pl.kernel(
        functools.partial(_edge_kernel, ngrp),
        mesh=mesh,
        out_type=jax.ShapeDtypeStruct((ep, 16), jnp.float32),
        scratch_types=[
            pltpu.VMEM((KSUB, CH), jnp.int32),
            pltpu.VMEM((KSUB, CH), jnp.int32),
            pltpu.VMEM((KSUB, CH, 16), jnp.float32),
            pltpu.VMEM((KSUB, CH, 16), jnp.float32),
            pltpu.VMEM((GRP, 16), jnp.float32),
            pltpu.SemaphoreType.DMA,
        ],
        compiler_params=pltpu.CompilerParams(use_tc_tiling_on_sc=False),
    )
    return fn(sjj2, djj2, a_tab, b_tab)


def _gate_pack(cp, nt, extra_bias_edges):
    ws, bs = [], []
    for g in GATES:
        ws.append(cp['W_' + g][nt])
        b = cp['b_' + g][nt]
        for en in extra_bias_edges:
            b = b + cp['conv_' + g][en]['lin_l_b'][None, :]
        bs.append(b)
    return jnp.concatenate(ws, axis=1), jnp.concatenate(bs, axis=1)


def _linl_pack(cp, en):
    return jnp.concatenate(
        [cp['conv_' + g][en]['lin_l_w'] for g in GATES], axis=1)


def _linr_pack(cp, ens):
    return jnp.concatenate(
        [sum(cp['conv_' + g][en]['lin_r_w'] for en in ens) for g in GATES],
        axis=1)


def _pad_edges(e, dump_row, grp=GRP):
    n = e.shape[1]
    npad = (-n) % grp
    src = jnp.pad(e[0], (0, npad))
    dst = jnp.pad(e[1], (0, npad), constant_values=dump_row)
    nc = (n + npad) // CH
    return src.reshape(nc, CH), dst.reshape(nc, CH)


def kernel(x_joint, x_grain, params, edge_index_jj, edge_index_jg,
           edge_index_gj):
    p = params
    nj = x_joint.shape[0]
    ng = x_grain.shape[0]
    ejj = edge_index_jj.shape[1]
    accr = ((nj + 16 * CH) // (16 * CH)) * 16 * CH
    xj = jnp.pad(x_joint, ((0, accr - nj), (0, 0)))

    we0j, be0j = _gate_pack(p['enc0'], 'joint', ['jj', 'gj'])
    we1j, be1j = _gate_pack(p['enc1'], 'joint', ['jj', 'gj'])
    we0g, be0g = _gate_pack(p['enc0'], 'grain', ['jg'])
    we1g, be1g = _gate_pack(p['enc1'], 'grain', ['jg'])
    h0a, h0b, h1a, h1b, c0, c1 = _enc_call(
        xj, we0j, be0j, we1j, be1j, True, 2048)
    g0a, g0b, g1a, g1b = _enc_call(
        x_grain, we0g, be0g, we1g, be1g, False, 2000)

    sjj2, djj2 = _pad_edges(edge_index_jj, nj)
    sgj2, dgj2 = _pad_edges(edge_index_gj, nj)
    (s_jj0a, s_jj1a, s_gj0a, s_gj1a, c_gj,
     s_jj0b, s_jj1b, s_gj0b, s_gj1b, c_jj) = _seg_call(
        nj, ng, accr, sjj2, djj2, sgj2, dgj2,
        (h0a, h0b, h1a, h1b), (g0a, g0b, g1a, g1b))

    wd0, bd0 = _gate_pack(p['dec0'], 'joint', ['jj', 'gj'])
    wd1, bd1 = _gate_pack(p['dec1'], 'joint', ['jj', 'gj'])
    mjj0 = _linl_pack(p['dec0'], 'jj')
    mgj0 = _linl_pack(p['dec0'], 'gj')
    hr0 = _linr_pack(p['dec0'], ['jj', 'gj'])
    mjj1 = _linl_pack(p['dec1'], 'jj')
    mgj1 = _linl_pack(p['dec1'], 'gj')
    hr1 = _linr_pack(p['dec1'], ['jj', 'gj'])
    za = jnp.zeros((OUT, 13), jnp.float32)
    wab = jnp.concatenate([p['lin2_w'][:OUT], p['lin1_w'][:OUT], za,
                           p['lin2_w'][OUT:], p['lin1_w'][OUT:], za], axis=1)
    bab = jnp.concatenate([p['lin2_b'], p['lin1_b'],
                           jnp.zeros((29,), jnp.float32)])[None, :]
    a_tab, b_tab = _dec_call(
        xj,
        [s_jj0a, s_jj0b, s_jj1a, s_jj1b,
         s_gj0a, s_gj0b, s_gj1a, s_gj1b],
        [c_jj, c_gj],
        [h0a, h0b, h1a, h1b], c0, c1,
        [wd0, mjj0, mgj0, hr0, bd0, wd1, mjj1, mgj1, hr1, bd1, wab, bab],
        2048)

    comb = _edge_call(sjj2, djj2, a_tab, b_tab)
    edge_event = comb[:ejj, 0]
    edge_rotation = comb[:ejj, 1:3]
    return edge_event, edge_rotation
